# Initial kernel scaffold; baseline (speedup 1.0000x reference)
import jax
import jax.numpy as jnp
from jax.experimental import pallas as pl


def kernel(x, edge_index, W1_l, b1_l, W1_r, W2_l, b2_l, W2_r):
    raise NotImplementedError("write your pallas kernel here")


# speedup vs baseline: 8.3216x; 8.3216x over previous
import functools

import jax
import jax.numpy as jnp
from jax import lax
from jax.experimental import pallas as pl
from jax.experimental.pallas import tpu as pltpu
from jax.experimental.pallas import tpu_sc as plsc

N_NODES = 10000
N_EDGES = 320000
D_IN = 128
D_H = 128
D_OUT = 64

NC = 2
NS = 16
NW = NC * NS
CHUNK1 = 128
CHUNK2 = 512
NFULL = N_NODES // 128
TAIL0 = NFULL * 128
TAILN = N_NODES - TAIL0


def _make_agg(d, chunk, with_deg):
    ksub = chunk // 128
    nch = N_EDGES // chunk
    mesh = plsc.VectorSubcoreMesh(core_axis_name="c", subcore_axis_name="s")
    out_type = [jax.ShapeDtypeStruct((NC, N_NODES, d), jnp.float32)]
    scratch = [
        pltpu.VMEM((ksub, 128), jnp.int32),
        pltpu.VMEM((ksub, 128), jnp.int32),
        pltpu.VMEM((chunk, d), jnp.float32),
        pltpu.VMEM_SHARED((N_NODES, d), jnp.float32),
        pltpu.SemaphoreType.DMA,
    ]
    if with_deg:
        out_type.append(jax.ShapeDtypeStruct((NC, N_NODES, 8), jnp.float32))
        scratch += [
            pltpu.VMEM((128, 8), jnp.float32),
            pltpu.VMEM((128, 8), jnp.float32),
            pltpu.VMEM_SHARED((N_NODES, 8), jnp.float32),
        ]

    @functools.partial(
        pl.kernel, mesh=mesh, out_type=out_type, scratch_types=scratch,
        compiler_params=pltpu.CompilerParams(use_tc_tiling_on_sc=False))
    def k(*refs):
        if with_deg:
            (y_hbm, src_hbm, dst_hbm, z_hbm, z8_hbm, ones_hbm,
             agg_out, deg_out,
             src_v, dst_v, rows_v, acc_sh, sem, ones_v, zd_v, deg_sh) = refs
        else:
            (y_hbm, src_hbm, dst_hbm, z_hbm,
             agg_out,
             src_v, dst_v, rows_v, acc_sh, sem) = refs
        cid = lax.axis_index("c")
        sid = lax.axis_index("s")
        w = cid * NS + sid

        pltpu.sync_copy(z_hbm, rows_v.at[pl.ds(0, 128)])
        if with_deg:
            pltpu.sync_copy(z8_hbm, zd_v)
            pltpu.sync_copy(ones_hbm, ones_v)
        nz = (NFULL - sid + NS - 1) // NS

        def zbody(i, carry):
            t = (sid + i * NS) * 128
            pltpu.sync_copy(rows_v.at[pl.ds(0, 128)],
                            acc_sh.at[pl.ds(t, 128)])
            if with_deg:
                pltpu.sync_copy(zd_v, deg_sh.at[pl.ds(t, 128)])
            return carry

        lax.fori_loop(0, nz, zbody, 0)

        @pl.when(sid == 0)
        def _():
            pltpu.sync_copy(rows_v.at[pl.ds(0, TAILN)],
                            acc_sh.at[pl.ds(TAIL0, TAILN)])
            if with_deg:
                pltpu.sync_copy(zd_v.at[pl.ds(0, TAILN)],
                                deg_sh.at[pl.ds(TAIL0, TAILN)])

        plsc.subcore_barrier()

        n_iter = (nch - w + NW - 1) // NW

        def body(i, carry):
            j = w + i * NW
            pltpu.sync_copy(src_hbm.at[j], src_v)
            pltpu.sync_copy(dst_hbm.at[j], dst_v)
            cps = [
                pltpu.async_copy(y_hbm.at[src_v.at[q]],
                                 rows_v.at[pl.ds(q * 128, 128)], sem)
                for q in range(ksub)
            ]
            for cp in cps:
                cp.wait()
            for q in range(ksub):
                pltpu.sync_copy(rows_v.at[pl.ds(q * 128, 128)],
                                acc_sh.at[dst_v.at[q]], add=True)
                if with_deg:
                    pltpu.sync_copy(ones_v, deg_sh.at[dst_v.at[q]],
                                    add=True)
            return carry

        lax.fori_loop(0, n_iter, body, 0)
        plsc.subcore_barrier()

        def wbody(i, carry):
            t = (sid + i * NS) * 128
            pltpu.sync_copy(acc_sh.at[pl.ds(t, 128)],
                            rows_v.at[pl.ds(0, 128)])
            pltpu.sync_copy(rows_v.at[pl.ds(0, 128)],
                            agg_out.at[cid].at[pl.ds(t, 128)])
            if with_deg:
                pltpu.sync_copy(deg_sh.at[pl.ds(t, 128)], zd_v)
                pltpu.sync_copy(zd_v, deg_out.at[cid].at[pl.ds(t, 128)])
            return carry

        lax.fori_loop(0, nz, wbody, 0)

        @pl.when(sid == 0)
        def _():
            pltpu.sync_copy(acc_sh.at[pl.ds(TAIL0, TAILN)],
                            rows_v.at[pl.ds(0, TAILN)])
            pltpu.sync_copy(rows_v.at[pl.ds(0, TAILN)],
                            agg_out.at[cid].at[pl.ds(TAIL0, TAILN)])
            if with_deg:
                pltpu.sync_copy(deg_sh.at[pl.ds(TAIL0, TAILN)],
                                zd_v.at[pl.ds(0, TAILN)])
                pltpu.sync_copy(zd_v.at[pl.ds(0, TAILN)],
                                deg_out.at[cid].at[pl.ds(TAIL0, TAILN)])

    return k


_agg_deg = _make_agg(D_H, CHUNK1, True)
_agg64 = _make_agg(D_OUT, CHUNK2, False)

_BN = 1000
_GRID = N_NODES // _BN


def _mm_body(x_ref, w_ref, o_ref):
    o_ref[...] = jnp.dot(x_ref[...], w_ref[...],
                         preferred_element_type=jnp.float32)


def _tc_mm(x, w):
    m, k = x.shape
    n = w.shape[1]
    return pl.pallas_call(
        _mm_body,
        grid=(_GRID,),
        in_specs=[
            pl.BlockSpec((_BN, k), lambda i: (i, 0)),
            pl.BlockSpec((k, n), lambda i: (0, 0)),
        ],
        out_specs=pl.BlockSpec((_BN, n), lambda i: (i, 0)),
        out_shape=jax.ShapeDtypeStruct((m, n), jnp.float32),
    )(x, w)


def _layer1_body(agg_ref, deg_ref, x_ref, wr_ref, b_ref, w2l_ref,
                 h_ref, y2_ref):
    agg = agg_ref[0] + agg_ref[1]
    d = deg_ref[...]
    deg = d[0, :, 0] + d[1, :, 0]
    inv = 1.0 / jnp.maximum(deg, 1.0)
    h = agg * inv[:, None] + b_ref[...] + jnp.dot(
        x_ref[...], wr_ref[...], preferred_element_type=jnp.float32)
    h = jnp.maximum(h, 0.0)
    h_ref[...] = h
    y2_ref[...] = jnp.dot(h, w2l_ref[...],
                          preferred_element_type=jnp.float32)


def _tc_layer1(agg, deg, x, w1r, b1, w2l):
    return pl.pallas_call(
        _layer1_body,
        grid=(_GRID,),
        in_specs=[
            pl.BlockSpec((NC, _BN, D_H), lambda i: (0, i, 0)),
            pl.BlockSpec((NC, _BN, 8), lambda i: (0, i, 0)),
            pl.BlockSpec((_BN, D_IN), lambda i: (i, 0)),
            pl.BlockSpec((D_IN, D_H), lambda i: (0, 0)),
            pl.BlockSpec((1, D_H), lambda i: (0, 0)),
            pl.BlockSpec((D_H, D_OUT), lambda i: (0, 0)),
        ],
        out_specs=[
            pl.BlockSpec((_BN, D_H), lambda i: (i, 0)),
            pl.BlockSpec((_BN, D_OUT), lambda i: (i, 0)),
        ],
        out_shape=[
            jax.ShapeDtypeStruct((N_NODES, D_H), jnp.float32),
            jax.ShapeDtypeStruct((N_NODES, D_OUT), jnp.float32),
        ],
    )(agg, deg, x, w1r, b1, w2l)


def _layer2_body(agg_ref, deg_ref, h_ref, wr_ref, b_ref, o_ref):
    agg = agg_ref[0] + agg_ref[1]
    d = deg_ref[...]
    deg = d[0, :, 0] + d[1, :, 0]
    inv = 1.0 / jnp.maximum(deg, 1.0)
    z = agg * inv[:, None] + b_ref[...] + jnp.dot(
        h_ref[...], wr_ref[...], preferred_element_type=jnp.float32)
    m = jnp.max(z, axis=1, keepdims=True)
    lse = jnp.log(jnp.sum(jnp.exp(z - m), axis=1, keepdims=True)) + m
    o_ref[...] = z - lse


def _tc_layer2(agg2, deg, h, w2r, b2):
    return pl.pallas_call(
        _layer2_body,
        grid=(_GRID,),
        in_specs=[
            pl.BlockSpec((NC, _BN, D_OUT), lambda i: (0, i, 0)),
            pl.BlockSpec((NC, _BN, 8), lambda i: (0, i, 0)),
            pl.BlockSpec((_BN, D_H), lambda i: (i, 0)),
            pl.BlockSpec((D_H, D_OUT), lambda i: (0, 0)),
            pl.BlockSpec((1, D_OUT), lambda i: (0, 0)),
        ],
        out_specs=pl.BlockSpec((_BN, D_OUT), lambda i: (i, 0)),
        out_shape=jax.ShapeDtypeStruct((N_NODES, D_OUT), jnp.float32),
    )(agg2, deg, h, w2r, b2)


def kernel(x, edge_index, W1_l, b1_l, W1_r, W2_l, b2_l, W2_r):
    ei = edge_index.astype(jnp.int32)
    src1 = ei[0].reshape(N_EDGES // CHUNK1, CHUNK1 // 128, 128)
    dst1 = ei[1].reshape(N_EDGES // CHUNK1, CHUNK1 // 128, 128)
    src2 = ei[0].reshape(N_EDGES // CHUNK2, CHUNK2 // 128, 128)
    dst2 = ei[1].reshape(N_EDGES // CHUNK2, CHUNK2 // 128, 128)
    z128 = jnp.zeros((128, D_H), jnp.float32)
    z64 = jnp.zeros((128, D_OUT), jnp.float32)
    z8 = jnp.zeros((128, 8), jnp.float32)
    ones8 = jnp.ones((128, 8), jnp.float32)

    y1 = _tc_mm(x, W1_l)
    agg1, deg = _agg_deg(y1, src1, dst1, z128, z8, ones8)
    h, y2 = _tc_layer1(agg1, deg, x, W1_r, b1_l.reshape(1, D_H), W2_l)
    (agg2,) = _agg64(y2, src2, dst2, z64)
    return _tc_layer2(agg2, deg, h, W2_r, b2_l.reshape(1, D_OUT))
